# Initial kernel scaffold; baseline (speedup 1.0000x reference)
import jax
import jax.numpy as jnp
from jax.experimental import pallas as pl


def kernel(q_sub, q_rel, hidden, edges, n_node, rela_embed, time_pe, Ws_W, Wr_W, Wqr_W, Wqr_b, fuse_W1, fuse_b1, fuse_W2, fuse_b2, wA, gate_gW, gate_gb, gate_hW, gate_hb, Wh):
    raise NotImplementedError("write your pallas kernel here")


# speedup vs baseline: 3.9869x; 3.9869x over previous
import functools

import jax
import jax.numpy as jnp
from jax import lax
from jax.experimental import pallas as pl
from jax.experimental.pallas import tpu as pltpu
from jax.experimental.pallas import tpu_sc as plsc

NC = 2
NS = 16
NW = NC * NS
CH = 128


def _wid():
    return lax.axis_index("s") * NC + lax.axis_index("c")


def _sc_mesh():
    return plsc.VectorSubcoreMesh(core_axis_name="c", subcore_axis_name="s")


def _build_hq_table(rela_embed, q_rel_pad):
    bp = q_rel_pad.shape[0]
    d = rela_embed.shape[1]
    per = bp // NW
    c0 = 64
    nck = per // c0

    @functools.partial(
        pl.kernel,
        mesh=_sc_mesh(),
        out_type=jax.ShapeDtypeStruct((bp, d), jnp.float32),
        scratch_types=[
            pltpu.VMEM((c0,), jnp.int32),
            pltpu.VMEM((c0, d), jnp.float32),
            pltpu.SemaphoreType.DMA,
        ],
    )
    def k(q_h, tab_h, out_h, idx_v, rows_v, sem):
        base = _wid() * per

        def body(j, carry):
            off = pl.multiple_of(base + j * c0, 8)
            pltpu.sync_copy(q_h.at[pl.ds(off, c0)], idx_v)
            pltpu.async_copy(tab_h.at[idx_v], rows_v, sem).wait()
            pltpu.sync_copy(rows_v, out_h.at[pl.ds(off, c0)])
            return carry

        lax.fori_loop(0, nck, body, 0)

    return k(q_rel_pad, rela_embed)


def _gather_edges(hidden, rela_embed, time_pe, hq_table, sub, r2, t6, r0):
    ep = sub.shape[0]
    per = ep // NW
    nck = per // CH
    d = hidden.shape[1]
    dt = time_pe.shape[1]

    @functools.partial(
        pl.kernel,
        mesh=_sc_mesh(),
        out_type=(
            jax.ShapeDtypeStruct((ep, d), jnp.float32),
            jax.ShapeDtypeStruct((ep, d), jnp.float32),
            jax.ShapeDtypeStruct((ep, dt), jnp.float32),
            jax.ShapeDtypeStruct((ep, d), jnp.float32),
        ),
        scratch_types=[
            pltpu.VMEM((CH,), jnp.int32),
            pltpu.VMEM((CH,), jnp.int32),
            pltpu.VMEM((CH,), jnp.int32),
            pltpu.VMEM((CH,), jnp.int32),
            pltpu.VMEM((CH, d), jnp.float32),
            pltpu.VMEM((CH, d), jnp.float32),
            pltpu.VMEM((CH, dt), jnp.float32),
            pltpu.VMEM((CH, d), jnp.float32),
            pltpu.SemaphoreType.DMA,
            pltpu.SemaphoreType.DMA,
            pltpu.SemaphoreType.DMA,
            pltpu.SemaphoreType.DMA,
        ],
    )
    def k(sub_h, r2_h, t6_h, r0_h, hid_h, rel_h, tpe_h, hqt_h,
          hs_o, rel_o, tpe_o, hq_o,
          i0, i1, i2, i3, b0, b1, b2, b3, s0, s1, s2, s3):
        base = _wid() * per

        def body(j, carry):
            off = pl.multiple_of(base + j * CH, 8)
            sl = pl.ds(off, CH)
            pltpu.sync_copy(sub_h.at[sl], i0)
            pltpu.sync_copy(r2_h.at[sl], i1)
            pltpu.sync_copy(t6_h.at[sl], i2)
            pltpu.sync_copy(r0_h.at[sl], i3)
            g0 = pltpu.async_copy(hid_h.at[i0], b0, s0)
            g1 = pltpu.async_copy(rel_h.at[i1], b1, s1)
            g2 = pltpu.async_copy(tpe_h.at[i2], b2, s2)
            g3 = pltpu.async_copy(hqt_h.at[i3], b3, s3)
            g0.wait()
            pltpu.sync_copy(b0, hs_o.at[sl])
            g1.wait()
            pltpu.sync_copy(b1, rel_o.at[sl])
            g2.wait()
            pltpu.sync_copy(b2, tpe_o.at[sl])
            g3.wait()
            pltpu.sync_copy(b3, hq_o.at[sl])
            return carry

        lax.fori_loop(0, nck, body, 0)

    return k(sub, r2, t6, r0, hidden, rela_embed, time_pe, hq_table)


def _scatter_segments(up, ee, obj, np_rows, z_up):
    ep = obj.shape[0]
    per = ep // NW
    nck = per // CH
    d = up.shape[1]
    zrows = np_rows // NS

    @functools.partial(
        pl.kernel,
        mesh=_sc_mesh(),
        out_type=(
            jax.ShapeDtypeStruct((NC, np_rows, d), jnp.float32),
            jax.ShapeDtypeStruct((NC, np_rows, d), jnp.float32),
        ),
        scratch_types=[
            pltpu.VMEM((CH,), jnp.int32),
            pltpu.VMEM((CH, d), jnp.float32),
            pltpu.VMEM_SHARED((np_rows, d), jnp.float32),
        ],
    )
    def k(up_h, ee_h, obj_h, zu_h, pu_o, pe_o, idx_v, bu, acc_u):
        sid = lax.axis_index("s")
        cid = lax.axis_index("c")
        base = (sid * NC + cid) * per
        zsl = pl.ds(sid * zrows, zrows)

        for src_h, out_o in ((up_h, pu_o), (ee_h, pe_o)):
            pltpu.sync_copy(zu_h, acc_u.at[zsl])
            plsc.subcore_barrier()

            def body(j, carry):
                off = pl.multiple_of(base + j * CH, 8)
                sl = pl.ds(off, CH)
                pltpu.sync_copy(obj_h.at[sl], idx_v)
                pltpu.sync_copy(src_h.at[sl], bu)
                pltpu.sync_copy(bu, acc_u.at[idx_v], add=True)
                return carry

            lax.fori_loop(0, nck, body, 0)
            plsc.subcore_barrier()
            pltpu.sync_copy(acc_u.at[zsl], out_o.at[cid, zsl])
            plsc.subcore_barrier()

    return k(up, ee, obj, z_up)


def _lrelu(x):
    return jnp.maximum(x, 0.01 * x)


def _edge_body(hs_r, rel_r, tpe_r, hq_r, w1a_r, w1b_r, b1_r, w2_r, b2_r,
               g1_r, g2_r, g3_r, gb_r, h1_r, h2_r, hb_r,
               wst_r, wrt_r, wqrt_r, qb_r, wa_r, up_r, ee_r):
    dot = lambda a, b: jnp.dot(a, b, preferred_element_type=jnp.float32)
    hs = hs_r[...]
    rel = rel_r[...]
    hq = hq_r[...]
    h1 = _lrelu(dot(rel, w1a_r[...]) + dot(tpe_r[...], w1b_r[...]) + b1_r[...])
    h2 = _lrelu(dot(h1, w2_r[...]) + b2_r[...])
    hr = h2 + rel
    g = jax.nn.sigmoid(dot(hr, g1_r[...]) + dot(hq, g2_r[...])
                       + dot(hs, g3_r[...]) + gb_r[...])
    d = hs.shape[1]
    upd = g[:, :d]
    rst = g[:, d:]
    cand = jnp.tanh(dot(hr, h1_r[...]) + dot(rst * hs, h2_r[...]) + hb_r[...])
    msg = (1.0 - upd) * hs + upd * cand
    al = _lrelu(dot(hs, wst_r[...]) + dot(hr, wrt_r[...])
                + dot(hq, wqrt_r[...]) + qb_r[...])
    a = jnp.sum(al * wa_r[...], axis=1, keepdims=True)
    e = jnp.exp(a)
    up_r[...] = e * msg
    ee_r[...] = jnp.broadcast_to(e, (e.shape[0], ee_r.shape[1]))


def _edge_compute(hs, rel, tpe, hq, w1a, w1b, b1, w2, b2,
                  g1, g2, g3, gb, h1w, h2w, hb, wst, wrt, wqrt, qb, wa):
    ep, d = hs.shape
    dt = tpe.shape[1]
    da = wst.shape[1]
    blk = 2048
    grid = ep // blk
    row = lambda n: pl.BlockSpec((blk, n), lambda i: (i, 0))
    full = lambda a: pl.BlockSpec(a.shape, lambda i: (0,) * a.ndim)
    return pl.pallas_call(
        _edge_body,
        grid=(grid,),
        in_specs=[row(d), row(d), row(dt), row(d)]
                 + [full(x) for x in (w1a, w1b, b1, w2, b2, g1, g2, g3, gb,
                                      h1w, h2w, hb, wst, wrt, wqrt, qb, wa)],
        out_specs=(row(d), row(d)),
        out_shape=(jax.ShapeDtypeStruct((ep, d), jnp.float32),
                   jax.ShapeDtypeStruct((ep, d), jnp.float32)),
    )(hs, rel, tpe, hq, w1a, w1b, b1, w2, b2, g1, g2, g3, gb,
      h1w, h2w, hb, wst, wrt, wqrt, qb, wa)


def _finish_body(pu_r, pe_r, wh_r, out_r):
    pu = pu_r[...]
    pe = pe_r[...]
    s = pu[0] + pu[1]
    b = pe[0, :, :1] + pe[1, :, :1] + 1e-5
    out_r[...] = jnp.dot(s / b, wh_r[...], preferred_element_type=jnp.float32)


def _finish(pu, pe, wh_t, n):
    d = pu.shape[2]
    de = pe.shape[2]
    blk = 2000
    grid = n // blk
    return pl.pallas_call(
        _finish_body,
        grid=(grid,),
        in_specs=[pl.BlockSpec((NC, blk, d), lambda i: (0, i, 0)),
                  pl.BlockSpec((NC, blk, de), lambda i: (0, i, 0)),
                  pl.BlockSpec(wh_t.shape, lambda i: (0, 0))],
        out_specs=pl.BlockSpec((blk, d), lambda i: (i, 0)),
        out_shape=jax.ShapeDtypeStruct((n, d), jnp.float32),
    )(pu, pe, wh_t)


def kernel(q_sub, q_rel, hidden, edges, n_node, rela_embed, time_pe,
           Ws_W, Wr_W, Wqr_W, Wqr_b, fuse_W1, fuse_b1, fuse_W2, fuse_b2,
           wA, gate_gW, gate_gb, gate_hW, gate_hb, Wh):
    e = edges.shape[0]
    n = hidden.shape[0]
    d = hidden.shape[1]

    step = NW * CH
    ep = ((e + step - 1) // step) * step
    pad = ep - e
    r0 = jnp.concatenate([edges[:, 0], jnp.zeros((pad,), jnp.int32)])
    r2 = jnp.concatenate([edges[:, 2], jnp.zeros((pad,), jnp.int32)])
    sub = jnp.concatenate([edges[:, 4], jnp.zeros((pad,), jnp.int32)])
    t6 = jnp.concatenate([edges[:, 6], jnp.zeros((pad,), jnp.int32)])

    b = q_rel.shape[0]
    bstep = 64 * NW
    bp = ((b + bstep - 1) // bstep) * bstep
    q_rel_pad = jnp.concatenate([q_rel, jnp.zeros((bp - b,), jnp.int32)])

    np_rows = ((n + 1 + NS * 8 - 1) // (NS * 8)) * (NS * 8)
    obj = jnp.concatenate([edges[:, 5], jnp.full((pad,), n, jnp.int32)])

    dt = time_pe.shape[1]
    time_pe_pad = jnp.pad(time_pe, ((0, 0), (0, d - dt)))

    hq_table = _build_hq_table(rela_embed, q_rel_pad)
    hs, rel, tpe, hq = _gather_edges(hidden, rela_embed, time_pe_pad, hq_table,
                                     sub, r2, t6, r0)

    w1a = fuse_W1[:, :d].T
    w1b = jnp.pad(fuse_W1[:, d:].T, ((0, d - dt), (0, 0)))
    gt = gate_gW.T
    ht = gate_hW.T
    up, ee = _edge_compute(
        hs, rel, tpe, hq,
        w1a, w1b, fuse_b1.reshape(1, -1), fuse_W2.T, fuse_b2.reshape(1, -1),
        gt[:d], gt[d:2 * d], gt[2 * d:], gate_gb.reshape(1, -1),
        ht[:d], ht[d:], gate_hb.reshape(1, -1),
        Ws_W.T, Wr_W.T, Wqr_W.T, Wqr_b.reshape(1, -1), wA)

    zrows = np_rows // NS
    z_up = jnp.zeros((zrows, d), jnp.float32)
    pu, pe = _scatter_segments(up, ee, obj, np_rows, z_up)
    return _finish(pu, pe, Wh.T, n)
